# baseline (speedup 1.0000x reference)
import functools

import jax
import jax.numpy as jnp
from jax import lax
from jax.experimental import pallas as pl
from jax.experimental.pallas import tpu as pltpu
from jax.experimental.pallas import tpu_sc as plsc

NC = 2
NS = 16
NW = NC * NS
LANE = 16
GR = 5


def _mesh():
    return plsc.VectorSubcoreMesh(
        core_axis_name="c", subcore_axis_name="s",
        num_cores=NC, num_subcores=NS)


def _offs(width):
    offs = list(range(0, width - LANE + 1, LANE))
    if width % LANE:
        offs.append(width - LANE)
    return offs


def _zero_rows(ref, nrows, width):
    z = jnp.zeros((LANE,), jnp.float32)
    for i in range(nrows):
        for off in _offs(width):
            ref[i, pl.ds(off, LANE)] = z


@functools.lru_cache(maxsize=None)
def _build_degree(N, NCH, CH):
    SPAN = 640
    K = 10
    assert NCH % K == 0

    PW = NCH * CH

    @functools.partial(
        pl.kernel,
        out_type=jax.ShapeDtypeStruct((NC, 2, N), jnp.float32),
        mesh=_mesh(),
        compiler_params=pltpu.CompilerParams(use_tc_tiling_on_sc=False),
        scratch_types=[
            pltpu.VMEM((PW,), jnp.int32),
            pltpu.VMEM((PW,), jnp.int32),
            pltpu.VMEM((CH,), jnp.float32),
            pltpu.VMEM((SPAN,), jnp.float32),
            pltpu.VMEM_SHARED((N,), jnp.float32),
            pltpu.VMEM_SHARED((N,), jnp.float32),
            pltpu.SemaphoreType.DMA,
        ],
    )
    def deg_kernel(src_hbm, dst_hbm, out_hbm,
                   src_v, dst_v, ones_v, zbuf, sh_do, sh_di, dsem):
        c = lax.axis_index("c")
        s = lax.axis_index("s")
        wid = s * NC + c
        ones = jnp.ones((LANE,), jnp.float32)
        zero = jnp.zeros((LANE,), jnp.float32)
        for off in _offs(CH):
            ones_v[pl.ds(off, LANE)] = ones
        for off in range(0, SPAN, LANE):
            zbuf[pl.ds(off, LANE)] = zero
        base = jnp.minimum(s * SPAN, N - SPAN)
        pltpu.sync_copy(zbuf, sh_do.at[pl.ds(base, SPAN)])
        pltpu.sync_copy(zbuf, sh_di.at[pl.ds(base, SPAN)])
        pltpu.sync_copy(src_hbm.at[pl.ds(wid * PW, PW)], src_v)
        pltpu.sync_copy(dst_hbm.at[pl.ds(wid * PW, PW)], dst_v)
        plsc.subcore_barrier()

        def body(i, carry):
            j0 = i * K
            for t in range(K):
                pltpu.async_copy(
                    ones_v, sh_do.at[src_v.at[pl.ds((j0 + t) * CH, CH)]],
                    dsem, add=True)
                pltpu.async_copy(
                    ones_v, sh_di.at[dst_v.at[pl.ds((j0 + t) * CH, CH)]],
                    dsem, add=True)
            for t in range(K):
                pltpu.make_async_copy(
                    ones_v, sh_do.at[src_v.at[pl.ds((j0 + t) * CH, CH)]],
                    dsem).wait()
                pltpu.make_async_copy(
                    ones_v, sh_di.at[dst_v.at[pl.ds((j0 + t) * CH, CH)]],
                    dsem).wait()
            return carry
        lax.fori_loop(0, NCH // K, body, None)
        plsc.subcore_barrier()

        pltpu.sync_copy(sh_do.at[pl.ds(base, SPAN)],
                        out_hbm.at[c, 0, pl.ds(base, SPAN)])
        pltpu.sync_copy(sh_di.at[pl.ds(base, SPAN)],
                        out_hbm.at[c, 1, pl.ds(base, SPAN)])

    return deg_kernel


@functools.lru_cache(maxsize=None)
def _build_prop(N, W, NCH, CH):
    RPT = N // NS
    ZR = 25
    NG = NCH // GR
    PW = NCH * CH
    assert RPT % ZR == 0 and NCH % (2 * GR) == 0 and NG >= 4

    @functools.partial(
        pl.kernel,
        out_type=jax.ShapeDtypeStruct((NC, N, W), jnp.float32),
        mesh=_mesh(),
        compiler_params=pltpu.CompilerParams(use_tc_tiling_on_sc=False),
        scratch_types=[
            pltpu.VMEM((PW,), jnp.int32),
            pltpu.VMEM((PW,), jnp.int32),
        ] + [pltpu.VMEM((CH, W), jnp.float32) for _ in range(2 * GR)] + [
            pltpu.VMEM((ZR, W), jnp.float32),
            pltpu.VMEM_SHARED((N, W), jnp.float32),
            pltpu.SemaphoreType.DMA,
            pltpu.SemaphoreType.DMA,
            pltpu.SemaphoreType.DMA,
            pltpu.SemaphoreType.DMA,
        ],
    )
    def prop_kernel(tab_hbm, src_hbm, dst_hbm, out_hbm,
                    src_v, dst_v, b0, b1, b2, b3, b4, b5, b6, b7, b8, b9,
                    zbuf, sh_agg, gsA, gsB, ssA, ssB):
        c = lax.axis_index("c")
        s = lax.axis_index("s")
        wid = s * NC + c
        bufs = ((b0, b1, b2, b3, b4), (b5, b6, b7, b8, b9))
        gsem = (gsA, gsB)
        ssem = (ssA, ssB)

        _zero_rows(zbuf, ZR, W)

        def zbody(j, carry):
            pltpu.sync_copy(zbuf, sh_agg.at[pl.ds(s * RPT + j * ZR, ZR)])
            return carry
        lax.fori_loop(0, RPT // ZR, zbody, None)

        pltpu.sync_copy(src_hbm.at[pl.ds(wid * PW, PW)], src_v)
        pltpu.sync_copy(dst_hbm.at[pl.ds(wid * PW, PW)], dst_v)
        plsc.subcore_barrier()

        def sidx(v, g, t):
            return v.at[pl.ds((g * GR + t) * CH, CH)]

        def fire_g(g, x):
            for t in range(GR):
                pltpu.async_copy(tab_hbm.at[sidx(src_v, g, t)],
                                 bufs[x][t], gsem[x])

        def drain_g(g, x):
            for t in range(GR):
                pltpu.make_async_copy(tab_hbm.at[sidx(src_v, g, t)],
                                      bufs[x][t], gsem[x]).wait()

        def fire_s(g, x):
            for t in range(GR):
                pltpu.async_copy(bufs[x][t],
                                 sh_agg.at[sidx(dst_v, g, t)],
                                 ssem[x], add=True)

        def drain_s(g, x):
            for t in range(GR):
                pltpu.make_async_copy(bufs[x][t],
                                      sh_agg.at[sidx(dst_v, g, t)],
                                      ssem[x]).wait()

        fire_g(0, 0)
        fire_g(1, 1)
        drain_g(0, 0)
        fire_s(0, 0)

        def body(i, carry):
            g = 2 * i + 1
            drain_s(g - 1, 0)
            fire_g(g + 1, 0)
            drain_g(g, 1)
            fire_s(g, 1)
            g2 = g + 1
            drain_s(g2 - 1, 1)
            fire_g(g2 + 1, 1)
            drain_g(g2, 0)
            fire_s(g2, 0)
            return carry
        lax.fori_loop(0, (NG - 2) // 2, body, None)

        gl = NG - 1
        drain_s(gl - 1, 0)
        drain_g(gl, 1)
        fire_s(gl, 1)
        drain_s(gl, 1)

        plsc.subcore_barrier()
        base = s * RPT
        pltpu.sync_copy(sh_agg.at[pl.ds(base, RPT)],
                        out_hbm.at[c, pl.ds(base, RPT)])

    return prop_kernel


def _tc1(x, W1, W2, degp, WP, B):
    N, F = x.shape
    H = W1.shape[1]
    C = W2.shape[1]
    G = N // B

    def body(x_ref, w1_ref, w2_ref, degp_ref, out_ref):
        xb = x_ref[...]
        wc = jnp.dot(w1_ref[...], w2_ref[...],
                     preferred_element_type=jnp.float32)
        z = jnp.dot(xb, wc, preferred_element_type=jnp.float32)
        dp = degp_ref[...]
        deg_out = dp[0, 0, :, 0] + dp[1, 0, :, 0]
        ns = lax.rsqrt(jnp.maximum(deg_out, 1.0))
        out_ref[:, :C] = z * ns[:, None]
        out_ref[:, C:C + 1] = ns[:, None]
        out_ref[:, C + 1:] = jnp.zeros((z.shape[0], WP - C - 1), jnp.float32)

    return pl.pallas_call(
        body,
        grid=(G,),
        in_specs=[
            pl.BlockSpec((B, F), lambda i: (i, 0)),
            pl.BlockSpec((F, H), lambda i: (0, 0)),
            pl.BlockSpec((H, C), lambda i: (0, 0)),
            pl.BlockSpec((NC, 2, B, 1), lambda i: (0, 0, i, 0)),
        ],
        out_specs=pl.BlockSpec((B, WP), lambda i: (i, 0)),
        out_shape=jax.ShapeDtypeStruct((N, WP), jnp.float32),
    )(x, W1, W2, degp)


def _tc2(p1, degp, C, B):
    _, N, WP = p1.shape
    G = N // B

    def body(p_ref, degp_ref, t2_ref, c_ref):
        p = p_ref[...]
        dp = degp_ref[...]
        deg_out = dp[0, 0, :, 0] + dp[1, 0, :, 0]
        deg_in = dp[0, 1, :, 0] + dp[1, 1, :, 0]
        ns = lax.rsqrt(jnp.maximum(deg_out, 1.0))
        nd = lax.rsqrt(jnp.maximum(deg_in, 1.0))
        ag = p[0] + p[1]
        y = nd[:, None] * ag[:, :C]
        t2_ref[...] = ns[:, None] * y
        c_ref[...] = (nd * ag[:, C])[:, None]

    return pl.pallas_call(
        body,
        grid=(G,),
        in_specs=[
            pl.BlockSpec((NC, B, WP), lambda i: (0, i, 0)),
            pl.BlockSpec((NC, 2, B, 1), lambda i: (0, 0, i, 0)),
        ],
        out_specs=[
            pl.BlockSpec((B, C), lambda i: (i, 0)),
            pl.BlockSpec((B, 1), lambda i: (i, 0)),
        ],
        out_shape=[
            jax.ShapeDtypeStruct((N, C), jnp.float32),
            jax.ShapeDtypeStruct((N, 1), jnp.float32),
        ],
    )(p1, degp)


def _tc3(p2, degp, cvec, b1r, W2, b2r, B):
    _, N, C = p2.shape
    H = W2.shape[0]
    G = N // B

    def body(q_ref, degp_ref, c_ref, b1_ref, w2_ref, b2_ref, out_ref):
        q = q_ref[...]
        dp = degp_ref[...]
        deg_in = dp[0, 1, :, 0] + dp[1, 1, :, 0]
        nd = lax.rsqrt(jnp.maximum(deg_in, 1.0))
        b1w2 = jnp.dot(b1_ref[...], w2_ref[...],
                       preferred_element_type=jnp.float32)
        logits = (nd[:, None] * (q[0] + q[1])
                  + c_ref[...] * b1w2 + b2_ref[...])
        m = jnp.max(logits, axis=1, keepdims=True)
        ex = jnp.exp(logits - m)
        lse = jnp.log(jnp.sum(ex, axis=1, keepdims=True)) + m
        out_ref[...] = logits - lse

    return pl.pallas_call(
        body,
        grid=(G,),
        in_specs=[
            pl.BlockSpec((NC, B, C), lambda i: (0, i, 0)),
            pl.BlockSpec((NC, 2, B, 1), lambda i: (0, 0, i, 0)),
            pl.BlockSpec((B, 1), lambda i: (i, 0)),
            pl.BlockSpec((1, H), lambda i: (0, 0)),
            pl.BlockSpec((H, C), lambda i: (0, 0)),
            pl.BlockSpec((1, C), lambda i: (0, 0)),
        ],
        out_specs=pl.BlockSpec((B, C), lambda i: (i, 0)),
        out_shape=jax.ShapeDtypeStruct((N, C), jnp.float32),
    )(p2, degp, cvec, b1r, W2, b2r)


def kernel(x, edge_index, W1, b1, W2, b2):
    N, F = x.shape
    C = W2.shape[1]
    E = edge_index.shape[1]
    per_w = E // NW
    assert per_w * NW == E and N % NS == 0

    CH = 40 if per_w % 40 == 0 else max(
        ch for ch in range(8, 129, 8)
        if per_w % ch == 0 and (per_w // ch) % (2 * GR) == 0)
    NCH = per_w // CH

    WP = ((C + 1 + 15) // 16) * 16
    B = 2000 if N % 2000 == 0 else N

    src_f = edge_index[0]
    dst_f = edge_index[1]

    degp = _build_degree(N, NCH, CH)(src_f, dst_f)
    degp = degp.reshape(NC, 2, N, 1)
    t1 = _tc1(x, W1, W2, degp, WP, B)
    p1 = _build_prop(N, WP, NCH, CH)(t1, src_f, dst_f)
    t2, cvec = _tc2(p1, degp, C, B)
    p2 = _build_prop(N, C, NCH, CH)(t2, src_f, dst_f)
    return _tc3(p2, degp, cvec, b1.reshape(1, -1), W2, b2.reshape(1, -1), B)


# speedup vs baseline: 1.1664x; 1.1664x over previous
import functools

import jax
import jax.numpy as jnp
from jax import lax
from jax.experimental import pallas as pl
from jax.experimental.pallas import tpu as pltpu
from jax.experimental.pallas import tpu_sc as plsc

NC = 2
NS = 16
NW = NC * NS
LANE = 16
GR = 4


def _mesh():
    return plsc.VectorSubcoreMesh(
        core_axis_name="c", subcore_axis_name="s",
        num_cores=NC, num_subcores=NS)


def _offs(width):
    offs = list(range(0, width - LANE + 1, LANE))
    if width % LANE:
        offs.append(width - LANE)
    return offs


def _zero_rows(ref, nrows, width):
    z = jnp.zeros((LANE,), jnp.float32)
    for i in range(nrows):
        for off in _offs(width):
            ref[i, pl.ds(off, LANE)] = z


@functools.lru_cache(maxsize=None)
def _build_degree(N, NCH, CH):
    SPAN = 640
    K = 8
    assert NCH % K == 0

    @functools.partial(
        pl.kernel,
        out_type=jax.ShapeDtypeStruct((NC, 2, N), jnp.float32),
        mesh=_mesh(),
        compiler_params=pltpu.CompilerParams(use_tc_tiling_on_sc=False),
        scratch_types=[
            pltpu.VMEM((NCH, CH), jnp.int32),
            pltpu.VMEM((NCH, CH), jnp.int32),
            pltpu.VMEM((CH,), jnp.float32),
            pltpu.VMEM((SPAN,), jnp.float32),
            pltpu.VMEM_SHARED((N,), jnp.float32),
            pltpu.VMEM_SHARED((N,), jnp.float32),
            pltpu.SemaphoreType.DMA,
        ],
    )
    def deg_kernel(src_hbm, dst_hbm, out_hbm,
                   src_v, dst_v, ones_v, zbuf, sh_do, sh_di, dsem):
        c = lax.axis_index("c")
        s = lax.axis_index("s")
        wid = s * NC + c
        ones = jnp.ones((LANE,), jnp.float32)
        zero = jnp.zeros((LANE,), jnp.float32)
        for off in _offs(CH):
            ones_v[pl.ds(off, LANE)] = ones
        for off in range(0, SPAN, LANE):
            zbuf[pl.ds(off, LANE)] = zero
        base = jnp.minimum(s * SPAN, N - SPAN)
        pltpu.sync_copy(zbuf, sh_do.at[pl.ds(base, SPAN)])
        pltpu.sync_copy(zbuf, sh_di.at[pl.ds(base, SPAN)])
        pltpu.sync_copy(src_hbm.at[wid], src_v)
        pltpu.sync_copy(dst_hbm.at[wid], dst_v)
        plsc.subcore_barrier()

        def body(i, carry):
            j0 = i * K
            for t in range(K):
                pltpu.async_copy(ones_v, sh_do.at[src_v.at[j0 + t]], dsem,
                                 add=True)
                pltpu.async_copy(ones_v, sh_di.at[dst_v.at[j0 + t]], dsem,
                                 add=True)
            for t in range(K):
                pltpu.make_async_copy(
                    ones_v, sh_do.at[src_v.at[j0 + t]], dsem).wait()
                pltpu.make_async_copy(
                    ones_v, sh_di.at[dst_v.at[j0 + t]], dsem).wait()
            return carry
        lax.fori_loop(0, NCH // K, body, None)
        plsc.subcore_barrier()

        pltpu.sync_copy(sh_do.at[pl.ds(base, SPAN)],
                        out_hbm.at[c, 0, pl.ds(base, SPAN)])
        pltpu.sync_copy(sh_di.at[pl.ds(base, SPAN)],
                        out_hbm.at[c, 1, pl.ds(base, SPAN)])

    return deg_kernel


@functools.lru_cache(maxsize=None)
def _build_prop(N, W, NCH, CH):
    RPT = N // NS
    ZR = 25
    NG = NCH // GR
    assert RPT % ZR == 0 and NCH % (2 * GR) == 0 and NG >= 4

    @functools.partial(
        pl.kernel,
        out_type=jax.ShapeDtypeStruct((NC, N, W), jnp.float32),
        mesh=_mesh(),
        compiler_params=pltpu.CompilerParams(use_tc_tiling_on_sc=False),
        scratch_types=[
            pltpu.VMEM((NCH, CH), jnp.int32),
            pltpu.VMEM((NCH, CH), jnp.int32),
        ] + [pltpu.VMEM((CH, W), jnp.float32) for _ in range(2 * GR)] + [
            pltpu.VMEM((ZR, W), jnp.float32),
            pltpu.VMEM_SHARED((N, W), jnp.float32),
            pltpu.SemaphoreType.DMA,
            pltpu.SemaphoreType.DMA,
            pltpu.SemaphoreType.DMA,
            pltpu.SemaphoreType.DMA,
        ],
    )
    def prop_kernel(tab_hbm, src_hbm, dst_hbm, out_hbm,
                    src_v, dst_v, b0, b1, b2, b3, b4, b5, b6, b7,
                    zbuf, sh_agg, gsA, gsB, ssA, ssB):
        c = lax.axis_index("c")
        s = lax.axis_index("s")
        wid = s * NC + c
        bufs = ((b0, b1, b2, b3), (b4, b5, b6, b7))
        gsem = (gsA, gsB)
        ssem = (ssA, ssB)

        _zero_rows(zbuf, ZR, W)

        def zbody(j, carry):
            pltpu.sync_copy(zbuf, sh_agg.at[pl.ds(s * RPT + j * ZR, ZR)])
            return carry
        lax.fori_loop(0, RPT // ZR, zbody, None)

        pltpu.sync_copy(src_hbm.at[wid], src_v)
        pltpu.sync_copy(dst_hbm.at[wid], dst_v)
        plsc.subcore_barrier()

        def fire_g(g, x):
            for t in range(GR):
                pltpu.async_copy(tab_hbm.at[src_v.at[g * GR + t]],
                                 bufs[x][t], gsem[x])

        def drain_g(g, x):
            for t in range(GR):
                pltpu.make_async_copy(tab_hbm.at[src_v.at[g * GR + t]],
                                      bufs[x][t], gsem[x]).wait()

        def fire_s(g, x):
            for t in range(GR):
                pltpu.async_copy(bufs[x][t],
                                 sh_agg.at[dst_v.at[g * GR + t]],
                                 ssem[x], add=True)

        def drain_s(g, x):
            for t in range(GR):
                pltpu.make_async_copy(bufs[x][t],
                                      sh_agg.at[dst_v.at[g * GR + t]],
                                      ssem[x]).wait()

        fire_g(0, 0)
        fire_g(1, 1)
        drain_g(0, 0)
        fire_s(0, 0)

        def body(i, carry):
            g = 2 * i + 1
            drain_s(g - 1, 0)
            fire_g(g + 1, 0)
            drain_g(g, 1)
            fire_s(g, 1)
            g2 = g + 1
            drain_s(g2 - 1, 1)
            fire_g(g2 + 1, 1)
            drain_g(g2, 0)
            fire_s(g2, 0)
            return carry
        lax.fori_loop(0, (NG - 2) // 2, body, None)

        gl = NG - 1
        drain_s(gl - 1, 0)
        drain_g(gl, 1)
        fire_s(gl, 1)
        drain_s(gl, 1)

        plsc.subcore_barrier()
        base = s * RPT
        pltpu.sync_copy(sh_agg.at[pl.ds(base, RPT)],
                        out_hbm.at[c, pl.ds(base, RPT)])

    return prop_kernel


def _tc_norms(degp):
    _, _, N = degp.shape

    def body(d_ref, out_ref):
        dp = d_ref[...]
        deg = dp[0] + dp[1]
        nrm = lax.rsqrt(jnp.maximum(deg, 1.0))
        out_ref[...] = nrm.T

    return pl.pallas_call(
        body,
        out_shape=jax.ShapeDtypeStruct((N, 2), jnp.float32),
    )(degp)


def _tc1(x, W1, W2, nrm, WP, B):
    N, F = x.shape
    H = W1.shape[1]
    C = W2.shape[1]
    G = N // B

    def body(x_ref, w1_ref, w2_ref, n_ref, out_ref):
        xb = x_ref[...]
        wc = jnp.dot(w1_ref[...], w2_ref[...],
                     preferred_element_type=jnp.float32)
        z = jnp.dot(xb, wc, preferred_element_type=jnp.float32)
        ns = n_ref[:, 0:1]
        out_ref[:, :C] = z * ns
        out_ref[:, C:C + 1] = ns
        out_ref[:, C + 1:] = jnp.zeros((z.shape[0], WP - C - 1), jnp.float32)

    return pl.pallas_call(
        body,
        grid=(G,),
        in_specs=[
            pl.BlockSpec((B, F), lambda i: (i, 0)),
            pl.BlockSpec((F, H), lambda i: (0, 0)),
            pl.BlockSpec((H, C), lambda i: (0, 0)),
            pl.BlockSpec((B, 2), lambda i: (i, 0)),
        ],
        out_specs=pl.BlockSpec((B, WP), lambda i: (i, 0)),
        out_shape=jax.ShapeDtypeStruct((N, WP), jnp.float32),
    )(x, W1, W2, nrm)


def _tc2(p1, nrm, C, B):
    _, N, WP = p1.shape
    G = N // B

    def body(p_ref, n_ref, t2_ref, c_ref):
        p = p_ref[...]
        ns = n_ref[:, 0:1]
        nd = n_ref[:, 1:2]
        ag = p[0] + p[1]
        t2_ref[...] = (ns * nd) * ag[:, :C]
        c_ref[...] = nd * ag[:, C:C + 1]

    return pl.pallas_call(
        body,
        grid=(G,),
        in_specs=[
            pl.BlockSpec((NC, B, WP), lambda i: (0, i, 0)),
            pl.BlockSpec((B, 2), lambda i: (i, 0)),
        ],
        out_specs=[
            pl.BlockSpec((B, C), lambda i: (i, 0)),
            pl.BlockSpec((B, 1), lambda i: (i, 0)),
        ],
        out_shape=[
            jax.ShapeDtypeStruct((N, C), jnp.float32),
            jax.ShapeDtypeStruct((N, 1), jnp.float32),
        ],
    )(p1, nrm)


def _tc3(p2, nrm, cvec, b1r, W2, b2r, B):
    _, N, C = p2.shape
    H = W2.shape[0]
    G = N // B

    def body(q_ref, n_ref, c_ref, b1_ref, w2_ref, b2_ref, out_ref):
        q = q_ref[...]
        nd = n_ref[:, 1:2]
        b1w2 = jnp.dot(b1_ref[...], w2_ref[...],
                       preferred_element_type=jnp.float32)
        logits = nd * (q[0] + q[1]) + c_ref[...] * b1w2 + b2_ref[...]
        m = jnp.max(logits, axis=1, keepdims=True)
        ex = jnp.exp(logits - m)
        lse = jnp.log(jnp.sum(ex, axis=1, keepdims=True)) + m
        out_ref[...] = logits - lse

    return pl.pallas_call(
        body,
        grid=(G,),
        in_specs=[
            pl.BlockSpec((NC, B, C), lambda i: (0, i, 0)),
            pl.BlockSpec((B, 2), lambda i: (i, 0)),
            pl.BlockSpec((B, 1), lambda i: (i, 0)),
            pl.BlockSpec((1, H), lambda i: (0, 0)),
            pl.BlockSpec((H, C), lambda i: (0, 0)),
            pl.BlockSpec((1, C), lambda i: (0, 0)),
        ],
        out_specs=pl.BlockSpec((B, C), lambda i: (i, 0)),
        out_shape=jax.ShapeDtypeStruct((N, C), jnp.float32),
    )(p2, nrm, cvec, b1r, W2, b2r)


def kernel(x, edge_index, W1, b1, W2, b2):
    N, F = x.shape
    C = W2.shape[1]
    E = edge_index.shape[1]
    per_w = E // NW
    assert per_w * NW == E and N % NS == 0

    CH = 125 if per_w % 125 == 0 else max(
        ch for ch in range(1, 129)
        if per_w % ch == 0 and (per_w // ch) % (2 * GR) == 0)
    NCH = per_w // CH

    WP = ((C + 1 + 15) // 16) * 16
    B = 2000 if N % 2000 == 0 else N

    src3 = edge_index[0].reshape(NW, NCH, CH)
    dst3 = edge_index[1].reshape(NW, NCH, CH)

    degp = _build_degree(N, NCH, CH)(src3, dst3)
    nrm = _tc_norms(degp)
    t1 = _tc1(x, W1, W2, nrm, WP, B)
    p1 = _build_prop(N, WP, NCH, CH)(t1, src3, dst3)
    t2, cvec = _tc2(p1, nrm, C, B)
    p2 = _build_prop(N, C, NCH, CH)(t2, src3, dst3)
    return _tc3(p2, nrm, cvec, b1.reshape(1, -1), W2, b2.reshape(1, -1), B)


# speedup vs baseline: 1.1703x; 1.0033x over previous
import functools

import jax
import jax.numpy as jnp
from jax import lax
from jax.experimental import pallas as pl
from jax.experimental.pallas import tpu as pltpu
from jax.experimental.pallas import tpu_sc as plsc

NC = 2
NS = 16
NW = NC * NS
LANE = 16
GR = 4


def _mesh():
    return plsc.VectorSubcoreMesh(
        core_axis_name="c", subcore_axis_name="s",
        num_cores=NC, num_subcores=NS)


def _offs(width):
    offs = list(range(0, width - LANE + 1, LANE))
    if width % LANE:
        offs.append(width - LANE)
    return offs


def _zero_rows(ref, nrows, width):
    z = jnp.zeros((LANE,), jnp.float32)
    for i in range(nrows):
        for off in _offs(width):
            ref[i, pl.ds(off, LANE)] = z


@functools.lru_cache(maxsize=None)
def _build_degree(N, NCH, CH):
    SPAN = 640
    K = 8
    assert NCH % K == 0

    @functools.partial(
        pl.kernel,
        out_type=jax.ShapeDtypeStruct((NC, 2, N), jnp.float32),
        mesh=_mesh(),
        compiler_params=pltpu.CompilerParams(use_tc_tiling_on_sc=False),
        scratch_types=[
            pltpu.VMEM((NCH, CH), jnp.int32),
            pltpu.VMEM((NCH, CH), jnp.int32),
            pltpu.VMEM((CH,), jnp.float32),
            pltpu.VMEM((SPAN,), jnp.float32),
            pltpu.VMEM_SHARED((N,), jnp.float32),
            pltpu.VMEM_SHARED((N,), jnp.float32),
            pltpu.SemaphoreType.DMA,
        ],
    )
    def deg_kernel(src_hbm, dst_hbm, out_hbm,
                   src_v, dst_v, ones_v, zbuf, sh_do, sh_di, dsem):
        c = lax.axis_index("c")
        s = lax.axis_index("s")
        wid = s * NC + c
        ones = jnp.ones((LANE,), jnp.float32)
        zero = jnp.zeros((LANE,), jnp.float32)
        for off in _offs(CH):
            ones_v[pl.ds(off, LANE)] = ones
        for off in range(0, SPAN, LANE):
            zbuf[pl.ds(off, LANE)] = zero
        base = jnp.minimum(s * SPAN, N - SPAN)
        pltpu.sync_copy(zbuf, sh_do.at[pl.ds(base, SPAN)])
        pltpu.sync_copy(zbuf, sh_di.at[pl.ds(base, SPAN)])
        pltpu.sync_copy(src_hbm.at[wid], src_v)
        pltpu.sync_copy(dst_hbm.at[wid], dst_v)
        plsc.subcore_barrier()

        def body(i, carry):
            j0 = i * K
            for t in range(K):
                pltpu.async_copy(ones_v, sh_do.at[src_v.at[j0 + t]], dsem,
                                 add=True)
                pltpu.async_copy(ones_v, sh_di.at[dst_v.at[j0 + t]], dsem,
                                 add=True)
            for t in range(K):
                pltpu.make_async_copy(
                    ones_v, sh_do.at[src_v.at[j0 + t]], dsem).wait()
                pltpu.make_async_copy(
                    ones_v, sh_di.at[dst_v.at[j0 + t]], dsem).wait()
            return carry
        lax.fori_loop(0, NCH // K, body, None)
        plsc.subcore_barrier()

        pltpu.sync_copy(sh_do.at[pl.ds(base, SPAN)],
                        out_hbm.at[c, 0, pl.ds(base, SPAN)])
        pltpu.sync_copy(sh_di.at[pl.ds(base, SPAN)],
                        out_hbm.at[c, 1, pl.ds(base, SPAN)])

    return deg_kernel


@functools.lru_cache(maxsize=None)
def _build_prop(N, W, NCH, CH):
    RPT = N // NS
    ZR = 25
    NG = NCH // GR
    assert RPT % ZR == 0 and NCH % (2 * GR) == 0 and NG >= 4

    @functools.partial(
        pl.kernel,
        out_type=jax.ShapeDtypeStruct((NC, N, W), jnp.float32),
        mesh=_mesh(),
        compiler_params=pltpu.CompilerParams(use_tc_tiling_on_sc=False),
        scratch_types=[
            pltpu.VMEM((NCH, CH), jnp.int32),
            pltpu.VMEM((NCH, CH), jnp.int32),
        ] + [pltpu.VMEM((CH, W), jnp.float32) for _ in range(2 * GR)] + [
            pltpu.VMEM((ZR, W), jnp.float32),
            pltpu.VMEM_SHARED((N, W), jnp.float32),
            pltpu.SemaphoreType.DMA,
            pltpu.SemaphoreType.DMA,
            pltpu.SemaphoreType.DMA,
            pltpu.SemaphoreType.DMA,
        ],
    )
    def prop_kernel(tab_hbm, src_hbm, dst_hbm, out_hbm,
                    src_v, dst_v, b0, b1, b2, b3, b4, b5, b6, b7,
                    zbuf, sh_agg, gsA, gsB, ssA, ssB):
        c = lax.axis_index("c")
        s = lax.axis_index("s")
        wid = s * NC + c
        bufs = ((b0, b1, b2, b3), (b4, b5, b6, b7))
        gsem = (gsA, gsB)
        ssem = (ssA, ssB)

        _zero_rows(zbuf, ZR, W)

        def zbody(j, carry):
            pltpu.sync_copy(zbuf, sh_agg.at[pl.ds(s * RPT + j * ZR, ZR)])
            return carry
        lax.fori_loop(0, RPT // ZR, zbody, None)

        pltpu.sync_copy(src_hbm.at[wid], src_v)
        pltpu.sync_copy(dst_hbm.at[wid], dst_v)
        plsc.subcore_barrier()

        def fire_g(g, x):
            for t in range(GR):
                pltpu.async_copy(tab_hbm.at[src_v.at[g * GR + t]],
                                 bufs[x][t], gsem[x])

        def drain_g(g, x):
            for t in range(GR):
                pltpu.make_async_copy(tab_hbm.at[src_v.at[g * GR + t]],
                                      bufs[x][t], gsem[x]).wait()

        def fire_s(g, x):
            for t in range(GR):
                pltpu.async_copy(bufs[x][t],
                                 sh_agg.at[dst_v.at[g * GR + t]],
                                 ssem[x], add=True)

        def drain_s(g, x):
            for t in range(GR):
                pltpu.make_async_copy(bufs[x][t],
                                      sh_agg.at[dst_v.at[g * GR + t]],
                                      ssem[x]).wait()

        fire_g(0, 0)
        fire_g(1, 1)
        drain_g(0, 0)
        fire_s(0, 0)

        def body(i, carry):
            g = 2 * i + 1
            drain_s(g - 1, 0)
            fire_g(g + 1, 0)
            drain_g(g, 1)
            fire_s(g, 1)
            g2 = g + 1
            drain_s(g2 - 1, 1)
            fire_g(g2 + 1, 1)
            drain_g(g2, 0)
            fire_s(g2, 0)
            return carry
        lax.fori_loop(0, (NG - 2) // 2, body, None)

        gl = NG - 1
        drain_s(gl - 1, 0)
        drain_g(gl, 1)
        fire_s(gl, 1)
        drain_s(gl, 1)

        plsc.subcore_barrier()
        base = s * RPT
        pltpu.sync_copy(sh_agg.at[pl.ds(base, RPT)],
                        out_hbm.at[c, pl.ds(base, RPT)])

    return prop_kernel


def _tc_norms(degp):
    _, _, N = degp.shape

    def body(d_ref, out_ref):
        dp = d_ref[...]
        deg = dp[0] + dp[1]
        nrm = lax.rsqrt(jnp.maximum(deg, 1.0))
        out_ref[...] = nrm.T

    return pl.pallas_call(
        body,
        out_shape=jax.ShapeDtypeStruct((N, 2), jnp.float32),
    )(degp)


def _tc1a(x, W1, W2, B):
    N, F = x.shape
    H = W1.shape[1]
    C = W2.shape[1]
    G = N // B

    def body(x_ref, w1_ref, w2_ref, out_ref):
        wc = jnp.dot(w1_ref[...], w2_ref[...],
                     preferred_element_type=jnp.float32)
        out_ref[...] = jnp.dot(x_ref[...], wc,
                               preferred_element_type=jnp.float32)

    return pl.pallas_call(
        body,
        grid=(G,),
        in_specs=[
            pl.BlockSpec((B, F), lambda i: (i, 0)),
            pl.BlockSpec((F, H), lambda i: (0, 0)),
            pl.BlockSpec((H, C), lambda i: (0, 0)),
        ],
        out_specs=pl.BlockSpec((B, C), lambda i: (i, 0)),
        out_shape=jax.ShapeDtypeStruct((N, C), jnp.float32),
    )(x, W1, W2)


def _tc1b(z, nrm, WP, B):
    N, C = z.shape
    G = N // B

    def body(z_ref, n_ref, out_ref):
        z = z_ref[...]
        ns = n_ref[:, 0:1]
        out_ref[:, :C] = z * ns
        out_ref[:, C:C + 1] = ns
        out_ref[:, C + 1:] = jnp.zeros((z.shape[0], WP - C - 1), jnp.float32)

    return pl.pallas_call(
        body,
        grid=(G,),
        in_specs=[
            pl.BlockSpec((B, C), lambda i: (i, 0)),
            pl.BlockSpec((B, 2), lambda i: (i, 0)),
        ],
        out_specs=pl.BlockSpec((B, WP), lambda i: (i, 0)),
        out_shape=jax.ShapeDtypeStruct((N, WP), jnp.float32),
    )(z, nrm)


def _tc2(p1, nrm, C, B):
    _, N, WP = p1.shape
    G = N // B

    def body(p_ref, n_ref, t2_ref, c_ref):
        p = p_ref[...]
        ns = n_ref[:, 0:1]
        nd = n_ref[:, 1:2]
        ag = p[0] + p[1]
        t2_ref[...] = (ns * nd) * ag[:, :C]
        c_ref[...] = nd * ag[:, C:C + 1]

    return pl.pallas_call(
        body,
        grid=(G,),
        in_specs=[
            pl.BlockSpec((NC, B, WP), lambda i: (0, i, 0)),
            pl.BlockSpec((B, 2), lambda i: (i, 0)),
        ],
        out_specs=[
            pl.BlockSpec((B, C), lambda i: (i, 0)),
            pl.BlockSpec((B, 1), lambda i: (i, 0)),
        ],
        out_shape=[
            jax.ShapeDtypeStruct((N, C), jnp.float32),
            jax.ShapeDtypeStruct((N, 1), jnp.float32),
        ],
    )(p1, nrm)


def _tc3(p2, nrm, cvec, b1r, W2, b2r, B):
    _, N, C = p2.shape
    H = W2.shape[0]
    G = N // B

    def body(q_ref, n_ref, c_ref, b1_ref, w2_ref, b2_ref, out_ref):
        q = q_ref[...]
        nd = n_ref[:, 1:2]
        b1w2 = jnp.dot(b1_ref[...], w2_ref[...],
                       preferred_element_type=jnp.float32)
        logits = nd * (q[0] + q[1]) + c_ref[...] * b1w2 + b2_ref[...]
        m = jnp.max(logits, axis=1, keepdims=True)
        ex = jnp.exp(logits - m)
        lse = jnp.log(jnp.sum(ex, axis=1, keepdims=True)) + m
        out_ref[...] = logits - lse

    return pl.pallas_call(
        body,
        grid=(G,),
        in_specs=[
            pl.BlockSpec((NC, B, C), lambda i: (0, i, 0)),
            pl.BlockSpec((B, 2), lambda i: (i, 0)),
            pl.BlockSpec((B, 1), lambda i: (i, 0)),
            pl.BlockSpec((1, H), lambda i: (0, 0)),
            pl.BlockSpec((H, C), lambda i: (0, 0)),
            pl.BlockSpec((1, C), lambda i: (0, 0)),
        ],
        out_specs=pl.BlockSpec((B, C), lambda i: (i, 0)),
        out_shape=jax.ShapeDtypeStruct((N, C), jnp.float32),
    )(p2, nrm, cvec, b1r, W2, b2r)


def kernel(x, edge_index, W1, b1, W2, b2):
    N, F = x.shape
    C = W2.shape[1]
    E = edge_index.shape[1]
    per_w = E // NW
    assert per_w * NW == E and N % NS == 0

    CH = 125 if per_w % 125 == 0 else max(
        ch for ch in range(1, 129)
        if per_w % ch == 0 and (per_w // ch) % (2 * GR) == 0)
    NCH = per_w // CH

    WP = ((C + 1 + 15) // 16) * 16
    B = 2000 if N % 2000 == 0 else N

    src3 = edge_index[0].reshape(NW, NCH, CH)
    dst3 = edge_index[1].reshape(NW, NCH, CH)

    z = _tc1a(x, W1, W2, B)
    degp = _build_degree(N, NCH, CH)(src3, dst3)
    nrm = _tc_norms(degp)
    t1 = _tc1b(z, nrm, WP, B)
    p1 = _build_prop(N, WP, NCH, CH)(t1, src3, dst3)
    t2, cvec = _tc2(p1, nrm, C, B)
    p2 = _build_prop(N, C, NCH, CH)(t2, src3, dst3)
    return _tc3(p2, nrm, cvec, b1.reshape(1, -1), W2, b2.reshape(1, -1), B)
